# baseline (speedup 1.0000x reference)
import jax
import jax.numpy as jnp
from jax.experimental import pallas as pl
from jax.experimental.pallas import tpu as pltpu

_B, _S, _D = 64, 512, 64
_H, _HD = 4, 16
_A, _M, _K = 32, 64, 8
_HID = 256
_NB = 4
_RT, _WT = 0.25, 0.25
_EPS = 1e-5
_NEG = -1e30
_CH = 128
_NC = _S // _CH

_bf = jnp.bfloat16


def _rms(h, w):
    return h * jax.lax.rsqrt(jnp.mean(h * h, axis=-1, keepdims=True) + _EPS) * w


def _l2n(v):
    n = jnp.sqrt(jnp.sum(v * v, axis=-1, keepdims=True))
    return v * jax.lax.reciprocal(jnp.maximum(n, 1e-12))


def _softmax(s):
    m = jnp.max(s, axis=-1, keepdims=True)
    e = jnp.exp(s - m)
    return e * jax.lax.reciprocal(jnp.sum(e, axis=-1, keepdims=True))


def _dot(a, b):
    return jnp.dot(a.astype(_bf), b.astype(_bf),
                   preferred_element_type=jnp.float32)


def _decoder_body(x_ref, mem_ref, attn_norm_ref, wq_ref, wk_ref, wv_ref, wo_ref,
                  read_norm_ref, read_q_ref, read_out_ref, ffn_norm_ref,
                  ffn_w1_ref, ffn_b1_ref, ffn_w2_ref, ffn_b2_ref, out_norm_ref,
                  write_norm_ref, mem_addr_ref, write_q_ref, write_v_ref,
                  gate_w_ref, gate_b_ref, out_x_ref, out_mem_ref):
    x = x_ref[0]
    mem = mem_ref[0]

    r = jax.lax.broadcasted_iota(jnp.int32, (_CH, _CH), 0)
    c = jax.lax.broadcasted_iota(jnp.int32, (_CH, _CH), 1)
    diag_allow = c <= r

    addr = _l2n(mem_addr_ref[...])
    mem_bf = mem.astype(_bf)

    for i in range(_NB):
        h = _rms(x, attn_norm_ref[i])
        q = _dot(h, wq_ref[i])
        k = _dot(h, wk_ref[i]).astype(_bf)
        v = _dot(h, wv_ref[i]).astype(_bf)
        o_rows = []
        for qi in range(_NC):
            w = (qi + 1) * _CH
            o_parts = []
            for hh in range(_H):
                sl = slice(hh * _HD, (hh + 1) * _HD)
                qc = q[qi * _CH:w, sl]
                s = jnp.dot(qc.astype(_bf), k[:w, sl].T,
                            preferred_element_type=jnp.float32) * 0.25
                if qi == 0:
                    s = jnp.where(diag_allow, s, _NEG)
                else:
                    s = jnp.concatenate(
                        [s[:, :qi * _CH],
                         jnp.where(diag_allow, s[:, qi * _CH:], _NEG)], axis=1)
                p = _softmax(s)
                o_parts.append(jnp.dot(p.astype(_bf), v[:w, sl],
                                       preferred_element_type=jnp.float32))
            o_rows.append(jnp.concatenate(o_parts, axis=1))
        o = jnp.concatenate(o_rows, axis=0)
        x = x + _dot(o, wo_ref[i])

        h = _rms(x, read_norm_ref[i])
        rq = _l2n(_dot(h, read_q_ref[i]))
        sc = _dot(rq, addr.T) * (1.0 / _RT)
        cur = sc
        for _ in range(_K):
            kth = jnp.max(cur, axis=-1, keepdims=True)
            cur = jnp.where(cur >= kth, _NEG, cur)
        p = _softmax(jnp.where(sc >= kth, sc, _NEG))
        rv = jnp.dot(p.astype(_bf), mem_bf,
                     preferred_element_type=jnp.float32)
        x = x + _dot(rv, read_out_ref[i])

        h = _rms(x, ffn_norm_ref[i])
        u = jax.nn.gelu(_dot(h, ffn_w1_ref[i]) + ffn_b1_ref[i])
        x = x + _dot(u, ffn_w2_ref[i]) + ffn_b2_ref[i]

    x = _rms(x, out_norm_ref[...])
    out_x_ref[0] = x

    h = _rms(x, write_norm_ref[...])
    wq2 = _l2n(_dot(h, write_q_ref[...]))
    sc = _dot(wq2, addr.T) * (1.0 / _WT)
    w = _softmax(sc)
    g = jax.nn.sigmoid(_dot(h, gate_w_ref[...]) + gate_b_ref[0, 0])
    w = w * g
    vu = _dot(h, write_v_ref[...])
    suw = jnp.sum(w, axis=0)
    sus = _dot(w.T, vu)
    upd = sus * jax.lax.reciprocal(jnp.maximum(suw, 1e-6))[:, None]
    sg = (1.0 - jnp.exp(-suw))[:, None]
    out_mem_ref[0] = mem * (1.0 - sg) + upd * sg


def _full(shape):
    n = len(shape)
    return pl.BlockSpec(shape, lambda b, _n=n: (0,) * _n)


def kernel(x, memory_values, attn_norm_w, wq, wk, wv, wo, read_norm_w, read_q,
           read_out, ffn_norm_w, ffn_w1, ffn_b1, ffn_w2, ffn_b2, out_norm_w,
           write_norm_w, mem_addr, write_q, write_v, gate_w, gate_b):
    gate_b2 = gate_b.reshape(1, 1)
    in_specs = [
        pl.BlockSpec((1, _S, _D), lambda b: (b, 0, 0)),
        pl.BlockSpec((1, _M, _D), lambda b: (b, 0, 0)),
        _full((_NB, _D)),
        _full((_NB, _D, _D)), _full((_NB, _D, _D)),
        _full((_NB, _D, _D)), _full((_NB, _D, _D)),
        _full((_NB, _D)),
        _full((_NB, _D, _A)), _full((_NB, _D, _D)),
        _full((_NB, _D)),
        _full((_NB, _D, _HID)), _full((_NB, _HID)),
        _full((_NB, _HID, _D)), _full((_NB, _D)),
        _full((_D,)),
        _full((_D,)),
        _full((_M, _A)),
        _full((_D, _A)), _full((_D, _D)),
        _full((_D, 1)), _full((1, 1)),
    ]
    out_specs = [
        pl.BlockSpec((1, _S, _D), lambda b: (b, 0, 0)),
        pl.BlockSpec((1, _M, _D), lambda b: (b, 0, 0)),
    ]
    out_shape = [
        jax.ShapeDtypeStruct((_B, _S, _D), jnp.float32),
        jax.ShapeDtypeStruct((_B, _M, _D), jnp.float32),
    ]
    out = pl.pallas_call(
        _decoder_body,
        grid=(_B,),
        in_specs=in_specs,
        out_specs=out_specs,
        out_shape=out_shape,
        compiler_params=pltpu.CompilerParams(
            dimension_semantics=("parallel",),
        ),
        name="scband_decoder",
    )(x, memory_values, attn_norm_w, wq, wk, wv, wo, read_norm_w, read_q,
      read_out, ffn_norm_w, ffn_w1, ffn_b1, ffn_w2, ffn_b2, out_norm_w,
      write_norm_w, mem_addr, write_q, write_v, gate_w, gate_b2)
    return (out[0], out[1])


# speedup vs baseline: 1.6297x; 1.6297x over previous
import jax
import jax.numpy as jnp
from jax.experimental import pallas as pl
from jax.experimental.pallas import tpu as pltpu

_B, _S, _D = 64, 512, 64
_H, _HD = 4, 16
_A, _M, _K = 32, 64, 8
_HID = 256
_NB = 4
_RT, _WT = 0.25, 0.25
_EPS = 1e-5
_NEG = -1e30
_R = 2
_G = _B // _R

_bf = jnp.bfloat16


def _rms(h, w):
    return h * jax.lax.rsqrt(jnp.mean(h * h, axis=-1, keepdims=True) + _EPS) * w


def _l2n(v):
    n = jnp.sqrt(jnp.sum(v * v, axis=-1, keepdims=True))
    return v * jax.lax.reciprocal(jnp.maximum(n, 1e-12))


def _softmax(s):
    m = jnp.max(s, axis=-1, keepdims=True)
    e = jnp.exp(s - m)
    return e * jax.lax.reciprocal(jnp.sum(e, axis=-1, keepdims=True))


def _dot(a, b):
    return jnp.dot(a.astype(_bf), b.astype(_bf),
                   preferred_element_type=jnp.float32)


def _decoder_body(x_ref, mem_ref, attn_norm_ref, wq_ref, wk_ref, wv_ref, wo_ref,
                  read_norm_ref, read_q_ref, read_out_ref, ffn_norm_ref,
                  ffn_w1_ref, ffn_b1_ref, ffn_w2_ref, ffn_b2_ref, out_norm_ref,
                  write_norm_ref, mem_addr_ref, write_q_ref, write_v_ref,
                  gate_w_ref, gate_b_ref, out_x_ref, out_mem_ref):
    x = x_ref[...].reshape(_R * _S, _D)

    r = jax.lax.broadcasted_iota(jnp.int32, (_S, _S), 0)
    c = jax.lax.broadcasted_iota(jnp.int32, (_S, _S), 1)
    allow = c <= r

    addr = _l2n(mem_addr_ref[...])
    mem_bf = [mem_ref[rr].astype(_bf) for rr in range(_R)]

    for i in range(_NB):
        h = _rms(x, attn_norm_ref[i])
        q = _dot(h, wq_ref[i])
        k = _dot(h, wk_ref[i]).astype(_bf)
        v = _dot(h, wv_ref[i]).astype(_bf)
        o_rows = []
        for rr in range(_R):
            row = slice(rr * _S, (rr + 1) * _S)
            o_parts = []
            for hh in range(_H):
                sl = slice(hh * _HD, (hh + 1) * _HD)
                s = jnp.dot(q[row, sl].astype(_bf), k[row, sl].T,
                            preferred_element_type=jnp.float32) * 0.25
                s = jnp.where(allow, s, _NEG)
                p = _softmax(s)
                o_parts.append(jnp.dot(p.astype(_bf), v[row, sl],
                                       preferred_element_type=jnp.float32))
            o_rows.append(jnp.concatenate(o_parts, axis=1))
        o = jnp.concatenate(o_rows, axis=0)
        x = x + _dot(o, wo_ref[i])

        h = _rms(x, read_norm_ref[i])
        rq = _l2n(_dot(h, read_q_ref[i]))
        sc = _dot(rq, addr.T) * (1.0 / _RT)
        cur = sc
        for _ in range(_K):
            kth = jnp.max(cur, axis=-1, keepdims=True)
            cur = jnp.where(cur >= kth, _NEG, cur)
        p = _softmax(jnp.where(sc >= kth, sc, _NEG)).astype(_bf)
        rv = jnp.concatenate(
            [jnp.dot(p[rr * _S:(rr + 1) * _S], mem_bf[rr],
                     preferred_element_type=jnp.float32) for rr in range(_R)],
            axis=0)
        x = x + _dot(rv, read_out_ref[i])

        h = _rms(x, ffn_norm_ref[i])
        u = jax.nn.gelu(_dot(h, ffn_w1_ref[i]) + ffn_b1_ref[i])
        x = x + _dot(u, ffn_w2_ref[i]) + ffn_b2_ref[i]

    x = _rms(x, out_norm_ref[...])
    out_x_ref[...] = x.reshape(_R, _S, _D)

    h = _rms(x, write_norm_ref[...])
    wq2 = _l2n(_dot(h, write_q_ref[...]))
    sc = _dot(wq2, addr.T) * (1.0 / _WT)
    w = _softmax(sc)
    g = jax.nn.sigmoid(_dot(h, gate_w_ref[...]) + gate_b_ref[0, 0])
    w = w * g
    vu = _dot(h, write_v_ref[...])
    for rr in range(_R):
        row = slice(rr * _S, (rr + 1) * _S)
        wr = w[row]
        suw = jnp.sum(wr, axis=0)
        sus = _dot(wr.T, vu[row])
        upd = sus * jax.lax.reciprocal(jnp.maximum(suw, 1e-6))[:, None]
        sg = (1.0 - jnp.exp(-suw))[:, None]
        out_mem_ref[rr] = mem_ref[rr] * (1.0 - sg) + upd * sg


def _full(shape):
    n = len(shape)
    return pl.BlockSpec(shape, lambda b, _n=n: (0,) * _n)


def kernel(x, memory_values, attn_norm_w, wq, wk, wv, wo, read_norm_w, read_q,
           read_out, ffn_norm_w, ffn_w1, ffn_b1, ffn_w2, ffn_b2, out_norm_w,
           write_norm_w, mem_addr, write_q, write_v, gate_w, gate_b):
    gate_b2 = gate_b.reshape(1, 1)
    in_specs = [
        pl.BlockSpec((_R, _S, _D), lambda b: (b, 0, 0)),
        pl.BlockSpec((_R, _M, _D), lambda b: (b, 0, 0)),
        _full((_NB, _D)),
        _full((_NB, _D, _D)), _full((_NB, _D, _D)),
        _full((_NB, _D, _D)), _full((_NB, _D, _D)),
        _full((_NB, _D)),
        _full((_NB, _D, _A)), _full((_NB, _D, _D)),
        _full((_NB, _D)),
        _full((_NB, _D, _HID)), _full((_NB, _HID)),
        _full((_NB, _HID, _D)), _full((_NB, _D)),
        _full((_D,)),
        _full((_D,)),
        _full((_M, _A)),
        _full((_D, _A)), _full((_D, _D)),
        _full((_D, 1)), _full((1, 1)),
    ]
    out_specs = [
        pl.BlockSpec((_R, _S, _D), lambda b: (b, 0, 0)),
        pl.BlockSpec((_R, _M, _D), lambda b: (b, 0, 0)),
    ]
    out_shape = [
        jax.ShapeDtypeStruct((_B, _S, _D), jnp.float32),
        jax.ShapeDtypeStruct((_B, _M, _D), jnp.float32),
    ]
    out = pl.pallas_call(
        _decoder_body,
        grid=(_G,),
        in_specs=in_specs,
        out_specs=out_specs,
        out_shape=out_shape,
        compiler_params=pltpu.CompilerParams(
            dimension_semantics=("parallel",),
        ),
        name="scband_decoder",
    )(x, memory_values, attn_norm_w, wq, wk, wv, wo, read_norm_w, read_q,
      read_out, ffn_norm_w, ffn_w1, ffn_b1, ffn_w2, ffn_b2, out_norm_w,
      write_norm_w, mem_addr, write_q, write_v, gate_w, gate_b2)
    return (out[0], out[1])


# speedup vs baseline: 1.6447x; 1.0092x over previous
import jax
import jax.numpy as jnp
from jax.experimental import pallas as pl
from jax.experimental.pallas import tpu as pltpu

_B, _S, _D = 64, 512, 64
_H, _HD = 4, 16
_A, _M, _K = 32, 64, 8
_HID = 256
_NB = 4
_RT, _WT = 0.25, 0.25
_EPS = 1e-5
_NEG = -1e30
_R = 2
_G = _B // _R
_CH = 256
_NC = _S // _CH

_bf = jnp.bfloat16


def _rms(h, w):
    return h * jax.lax.rsqrt(jnp.mean(h * h, axis=-1, keepdims=True) + _EPS) * w


def _l2n(v):
    n = jnp.sqrt(jnp.sum(v * v, axis=-1, keepdims=True))
    return v * jax.lax.reciprocal(jnp.maximum(n, 1e-12))


def _softmax(s):
    m = jnp.max(s, axis=-1, keepdims=True)
    e = jnp.exp(s - m)
    return e * jax.lax.reciprocal(jnp.sum(e, axis=-1, keepdims=True))


def _dot(a, b):
    return jnp.dot(a.astype(_bf), b.astype(_bf),
                   preferred_element_type=jnp.float32)


def _decoder_body(x_ref, mem_ref, attn_norm_ref, wq_ref, wk_ref, wv_ref, wo_ref,
                  read_norm_ref, read_q_ref, read_out_ref, ffn_norm_ref,
                  ffn_w1_ref, ffn_b1_ref, ffn_w2_ref, ffn_b2_ref, out_norm_ref,
                  write_norm_ref, mem_addr_ref, write_q_ref, write_v_ref,
                  gate_w_ref, gate_b_ref, out_x_ref, out_mem_ref):
    x = x_ref[...].reshape(_R * _S, _D)

    r = jax.lax.broadcasted_iota(jnp.int32, (_S, _S), 0)
    c = jax.lax.broadcasted_iota(jnp.int32, (_S, _S), 1)
    allow = c <= r

    addr = _l2n(mem_addr_ref[...])
    mem_bf = [mem_ref[rr].astype(_bf) for rr in range(_R)]

    for i in range(_NB):
        h = _rms(x, attn_norm_ref[i])
        q = _dot(h, wq_ref[i] * 0.25)
        k = _dot(h, wk_ref[i]).astype(_bf)
        v = _dot(h, wv_ref[i]).astype(_bf)
        o_rows = []
        for rr in range(_R):
            row = slice(rr * _S, (rr + 1) * _S)
            o_parts = []
            for hh in range(_H):
                sl = slice(hh * _HD, (hh + 1) * _HD)
                s = jnp.dot(q[row, sl].astype(_bf), k[row, sl].T,
                            preferred_element_type=jnp.float32)
                s = jnp.where(allow, s, _NEG)
                m = jnp.max(s, axis=-1, keepdims=True)
                e = jnp.exp(s - m)
                ov = jnp.dot(e.astype(_bf), v[row, sl],
                             preferred_element_type=jnp.float32)
                o_parts.append(
                    ov * jax.lax.reciprocal(jnp.sum(e, axis=-1, keepdims=True)))
            o_rows.append(jnp.concatenate(o_parts, axis=1))
        o = jnp.concatenate(o_rows, axis=0)
        x = x + _dot(o, wo_ref[i])

        h = _rms(x, read_norm_ref[i])
        rq = _dot(h, read_q_ref[i])
        inv_n = jax.lax.reciprocal(
            jnp.maximum(jnp.sqrt(jnp.sum(rq * rq, axis=-1)), 1e-12))
        sc = jax.lax.dot_general(
            addr.astype(_bf), rq.astype(_bf),
            (((1,), (1,)), ((), ())),
            preferred_element_type=jnp.float32) * (inv_n * (1.0 / _RT))[None, :]
        cur = sc
        for _ in range(_K):
            kth = jnp.max(cur, axis=0, keepdims=True)
            cur = jnp.where(cur >= kth, _NEG, cur)
        sc = jnp.where(sc >= kth, sc, _NEG)
        m0 = jnp.max(sc, axis=0, keepdims=True)
        e = jnp.exp(sc - m0)
        inv_z = jax.lax.reciprocal(jnp.sum(e, axis=0))
        e = e.astype(_bf)
        rv = jnp.concatenate(
            [jax.lax.dot_general(
                e[:, rr * _S:(rr + 1) * _S], mem_bf[rr],
                (((0,), (0,)), ((), ())),
                preferred_element_type=jnp.float32) for rr in range(_R)],
            axis=0) * inv_z[:, None]
        x = x + _dot(rv, read_out_ref[i])

        h = _rms(x, ffn_norm_ref[i])
        u = jax.nn.gelu(_dot(h, ffn_w1_ref[i]) + ffn_b1_ref[i])
        x = x + _dot(u, ffn_w2_ref[i]) + ffn_b2_ref[i]

    x = _rms(x, out_norm_ref[...])
    out_x_ref[...] = x.reshape(_R, _S, _D)

    h = _rms(x, write_norm_ref[...])
    wq2 = _l2n(_dot(h, write_q_ref[...]))
    sc = _dot(wq2, addr.T) * (1.0 / _WT)
    w = _softmax(sc)
    g = jax.nn.sigmoid(_dot(h, gate_w_ref[...]) + gate_b_ref[0, 0])
    w = w * g
    vu = _dot(h, write_v_ref[...])
    for rr in range(_R):
        row = slice(rr * _S, (rr + 1) * _S)
        wr = w[row]
        suw = jnp.sum(wr, axis=0)
        sus = _dot(wr.T, vu[row])
        upd = sus * jax.lax.reciprocal(jnp.maximum(suw, 1e-6))[:, None]
        sg = (1.0 - jnp.exp(-suw))[:, None]
        out_mem_ref[rr] = mem_ref[rr] * (1.0 - sg) + upd * sg


def _full(shape):
    n = len(shape)
    return pl.BlockSpec(shape, lambda b, _n=n: (0,) * _n)


def kernel(x, memory_values, attn_norm_w, wq, wk, wv, wo, read_norm_w, read_q,
           read_out, ffn_norm_w, ffn_w1, ffn_b1, ffn_w2, ffn_b2, out_norm_w,
           write_norm_w, mem_addr, write_q, write_v, gate_w, gate_b):
    gate_b2 = gate_b.reshape(1, 1)
    in_specs = [
        pl.BlockSpec((_R, _S, _D), lambda b: (b, 0, 0)),
        pl.BlockSpec((_R, _M, _D), lambda b: (b, 0, 0)),
        _full((_NB, _D)),
        _full((_NB, _D, _D)), _full((_NB, _D, _D)),
        _full((_NB, _D, _D)), _full((_NB, _D, _D)),
        _full((_NB, _D)),
        _full((_NB, _D, _A)), _full((_NB, _D, _D)),
        _full((_NB, _D)),
        _full((_NB, _D, _HID)), _full((_NB, _HID)),
        _full((_NB, _HID, _D)), _full((_NB, _D)),
        _full((_D,)),
        _full((_D,)),
        _full((_M, _A)),
        _full((_D, _A)), _full((_D, _D)),
        _full((_D, 1)), _full((1, 1)),
    ]
    out_specs = [
        pl.BlockSpec((_R, _S, _D), lambda b: (b, 0, 0)),
        pl.BlockSpec((_R, _M, _D), lambda b: (b, 0, 0)),
    ]
    out_shape = [
        jax.ShapeDtypeStruct((_B, _S, _D), jnp.float32),
        jax.ShapeDtypeStruct((_B, _M, _D), jnp.float32),
    ]
    out = pl.pallas_call(
        _decoder_body,
        grid=(_G,),
        in_specs=in_specs,
        out_specs=out_specs,
        out_shape=out_shape,
        compiler_params=pltpu.CompilerParams(
            dimension_semantics=("parallel",),
        ),
        name="scband_decoder",
    )(x, memory_values, attn_norm_w, wq, wk, wv, wo, read_norm_w, read_q,
      read_out, ffn_norm_w, ffn_w1, ffn_b1, ffn_w2, ffn_b2, out_norm_w,
      write_norm_w, mem_addr, write_q, write_v, gate_w, gate_b2)
    return (out[0], out[1])


# speedup vs baseline: 1.7273x; 1.0502x over previous
import jax
import jax.numpy as jnp
from jax.experimental import pallas as pl
from jax.experimental.pallas import tpu as pltpu

_B, _S, _D = 64, 512, 64
_H, _HD = 4, 16
_A, _M, _K = 32, 64, 8
_HID = 256
_NB = 4
_RT, _WT = 0.25, 0.25
_EPS = 1e-5
_NEG = -1e30
_R = 2
_G = _B // _R
_CH = 256
_NC = _S // _CH

_bf = jnp.bfloat16


def _rms(h, w):
    return h * jax.lax.rsqrt(jnp.mean(h * h, axis=-1, keepdims=True) + _EPS) * w


def _l2n(v):
    n = jnp.sqrt(jnp.sum(v * v, axis=-1, keepdims=True))
    return v * jax.lax.reciprocal(jnp.maximum(n, 1e-12))


def _softmax(s):
    m = jnp.max(s, axis=-1, keepdims=True)
    e = jnp.exp(s - m)
    return e * jax.lax.reciprocal(jnp.sum(e, axis=-1, keepdims=True))


def _dot(a, b):
    return jnp.dot(a.astype(_bf), b.astype(_bf),
                   preferred_element_type=jnp.float32)


def _decoder_body(x_ref, mem_ref, attn_norm_ref, wq_ref, wk_ref, wv_ref, wo_ref,
                  read_norm_ref, read_q_ref, read_out_ref, ffn_norm_ref,
                  ffn_w1_ref, ffn_b1_ref, ffn_w2_ref, ffn_b2_ref, out_norm_ref,
                  write_norm_ref, mem_addr_ref, write_q_ref, write_v_ref,
                  gate_w_ref, gate_b_ref, out_x_ref, out_mem_ref):
    x = x_ref[...].reshape(_R * _S, _D)

    r = jax.lax.broadcasted_iota(jnp.int32, (_S, _S), 0)
    c = jax.lax.broadcasted_iota(jnp.int32, (_S, _S), 1)
    allow = c <= r

    addr = _l2n(mem_addr_ref[...])
    mem_bf = [mem_ref[rr].astype(_bf) for rr in range(_R)]
    ones_col = jnp.ones((_S, 1), _bf)

    for i in range(_NB):
        h = _rms(x, attn_norm_ref[i])
        q = _dot(h, wq_ref[i] * 0.25)
        k = _dot(h, wk_ref[i]).astype(_bf)
        v = _dot(h, wv_ref[i]).astype(_bf)
        o_rows = []
        for rr in range(_R):
            row = slice(rr * _S, (rr + 1) * _S)
            o_parts = []
            for hh in range(_H):
                sl = slice(hh * _HD, (hh + 1) * _HD)
                s = jnp.dot(q[row, sl].astype(_bf), k[row, sl].T,
                            preferred_element_type=jnp.float32)
                s = jnp.where(allow, s, _NEG)
                m = jnp.max(s, axis=-1, keepdims=True)
                e = jnp.exp(s - m).astype(_bf)
                ve = jnp.concatenate([v[row, sl], ones_col], axis=1)
                ov = jnp.dot(e, ve, preferred_element_type=jnp.float32)
                o_parts.append(
                    ov[:, :_HD] * jax.lax.reciprocal(ov[:, _HD:]))
            o_rows.append(jnp.concatenate(o_parts, axis=1))
        o = jnp.concatenate(o_rows, axis=0)
        x = x + _dot(o, wo_ref[i])

        h = _rms(x, read_norm_ref[i])
        rq = _dot(h, read_q_ref[i])
        inv_n = jax.lax.reciprocal(
            jnp.maximum(jnp.sqrt(jnp.sum(rq * rq, axis=-1)), 1e-12))
        sc = jax.lax.dot_general(
            addr.astype(_bf), rq.astype(_bf),
            (((1,), (1,)), ((), ())),
            preferred_element_type=jnp.float32) * (inv_n * (1.0 / _RT))[None, :]
        cur = sc
        for _ in range(_K):
            kth = jnp.max(cur, axis=0, keepdims=True)
            cur = jnp.where(cur >= kth, _NEG, cur)
        sc = jnp.where(sc >= kth, sc, _NEG)
        m0 = jnp.max(sc, axis=0, keepdims=True)
        e = jnp.exp(sc - m0)
        inv_z = jax.lax.reciprocal(jnp.sum(e, axis=0))
        e = e.astype(_bf)
        rv = jnp.concatenate(
            [jax.lax.dot_general(
                e[:, rr * _S:(rr + 1) * _S], mem_bf[rr],
                (((0,), (0,)), ((), ())),
                preferred_element_type=jnp.float32) for rr in range(_R)],
            axis=0) * inv_z[:, None]
        x = x + _dot(rv, read_out_ref[i])

        h = _rms(x, ffn_norm_ref[i])
        u = jax.nn.gelu(_dot(h, ffn_w1_ref[i]) + ffn_b1_ref[i])
        x = x + _dot(u, ffn_w2_ref[i]) + ffn_b2_ref[i]

    x = _rms(x, out_norm_ref[...])
    out_x_ref[...] = x.reshape(_R, _S, _D)

    h = _rms(x, write_norm_ref[...])
    wq2 = _l2n(_dot(h, write_q_ref[...]))
    sc = _dot(wq2, addr.T) * (1.0 / _WT)
    w = _softmax(sc)
    g = jax.nn.sigmoid(_dot(h, gate_w_ref[...]) + gate_b_ref[0, 0])
    w = w * g
    vu = _dot(h, write_v_ref[...])
    for rr in range(_R):
        row = slice(rr * _S, (rr + 1) * _S)
        wr = w[row]
        suw = jnp.sum(wr, axis=0)
        sus = _dot(wr.T, vu[row])
        upd = sus * jax.lax.reciprocal(jnp.maximum(suw, 1e-6))[:, None]
        sg = (1.0 - jnp.exp(-suw))[:, None]
        out_mem_ref[rr] = mem_ref[rr] * (1.0 - sg) + upd * sg


def _full(shape):
    n = len(shape)
    return pl.BlockSpec(shape, lambda b, _n=n: (0,) * _n)


def kernel(x, memory_values, attn_norm_w, wq, wk, wv, wo, read_norm_w, read_q,
           read_out, ffn_norm_w, ffn_w1, ffn_b1, ffn_w2, ffn_b2, out_norm_w,
           write_norm_w, mem_addr, write_q, write_v, gate_w, gate_b):
    gate_b2 = gate_b.reshape(1, 1)
    in_specs = [
        pl.BlockSpec((_R, _S, _D), lambda b: (b, 0, 0)),
        pl.BlockSpec((_R, _M, _D), lambda b: (b, 0, 0)),
        _full((_NB, _D)),
        _full((_NB, _D, _D)), _full((_NB, _D, _D)),
        _full((_NB, _D, _D)), _full((_NB, _D, _D)),
        _full((_NB, _D)),
        _full((_NB, _D, _A)), _full((_NB, _D, _D)),
        _full((_NB, _D)),
        _full((_NB, _D, _HID)), _full((_NB, _HID)),
        _full((_NB, _HID, _D)), _full((_NB, _D)),
        _full((_D,)),
        _full((_D,)),
        _full((_M, _A)),
        _full((_D, _A)), _full((_D, _D)),
        _full((_D, 1)), _full((1, 1)),
    ]
    out_specs = [
        pl.BlockSpec((_R, _S, _D), lambda b: (b, 0, 0)),
        pl.BlockSpec((_R, _M, _D), lambda b: (b, 0, 0)),
    ]
    out_shape = [
        jax.ShapeDtypeStruct((_B, _S, _D), jnp.float32),
        jax.ShapeDtypeStruct((_B, _M, _D), jnp.float32),
    ]
    out = pl.pallas_call(
        _decoder_body,
        grid=(_G,),
        in_specs=in_specs,
        out_specs=out_specs,
        out_shape=out_shape,
        compiler_params=pltpu.CompilerParams(
            dimension_semantics=("parallel",),
        ),
        name="scband_decoder",
    )(x, memory_values, attn_norm_w, wq, wk, wv, wo, read_norm_w, read_q,
      read_out, ffn_norm_w, ffn_w1, ffn_b1, ffn_w2, ffn_b2, out_norm_w,
      write_norm_w, mem_addr, write_q, write_v, gate_w, gate_b2)
    return (out[0], out[1])
